# Initial kernel scaffold; baseline (speedup 1.0000x reference)
import jax
import jax.numpy as jnp
from jax import lax
from jax.experimental import pallas as pl
from jax.experimental.pallas import tpu as pltpu
from jax.experimental.pallas import tpu_sc as plsc

_NUM_ATOMS = 320000
_FEA = 128
_NUM_SEG = 10000
_NC = 2
_NS = 16
_COLS = _FEA // _NC
_APT = _NUM_ATOMS // _NS
_CHUNK = 80
_NCHUNK = _APT // _CHUNK
_ROWS_PT = _NUM_SEG // _NS
_ZROWS = 125


def _body(feas, owner, out, rbuf, obuf, zbuf, acc):
    c = lax.axis_index("c")
    s = lax.axis_index("s")
    col0 = c * _COLS
    base = s * _APT

    def _zrow(i, carry):
        for j in range(_COLS // 16):
            zbuf[i, pl.ds(j * 16, 16)] = jnp.zeros((16,), jnp.float32)
        return carry

    lax.fori_loop(0, _ZROWS, _zrow, 0)
    r0 = s * _ROWS_PT
    for j in range(_ROWS_PT // _ZROWS):
        pltpu.sync_copy(zbuf, acc.at[pl.ds(r0 + j * _ZROWS, _ZROWS), :])
    plsc.subcore_barrier()

    def _chunk(k, carry):
        a0 = base + k * _CHUNK
        pltpu.sync_copy(feas.at[pl.ds(a0, _CHUNK), pl.ds(col0, _COLS)], rbuf)
        pltpu.sync_copy(owner.at[pl.ds(a0, _CHUNK)], obuf)
        pltpu.sync_copy(rbuf, acc.at[obuf], add=True)
        return carry

    lax.fori_loop(0, _NCHUNK, _chunk, 0)
    plsc.subcore_barrier()

    pltpu.sync_copy(
        acc.at[pl.ds(r0, _ROWS_PT), :],
        out.at[pl.ds(r0, _ROWS_PT), pl.ds(col0, _COLS)],
    )


_pool = pl.kernel(
    _body,
    out_type=jax.ShapeDtypeStruct((_NUM_SEG, _FEA), jnp.float32),
    mesh=plsc.VectorSubcoreMesh(
        core_axis_name="c", subcore_axis_name="s", num_cores=_NC,
        num_subcores=_NS,
    ),
    scratch_types=[
        pltpu.VMEM((_CHUNK, _COLS), jnp.float32),
        pltpu.VMEM((_CHUNK,), jnp.int32),
        pltpu.VMEM((_ZROWS, _COLS), jnp.float32),
        pltpu.VMEM_SHARED((_NUM_SEG, _COLS), jnp.float32),
    ],
)


@jax.jit
def kernel(atom_feas, atom_owner):
    return _pool(atom_feas, atom_owner.astype(jnp.int32))


# speedup vs baseline: 2.6661x; 2.6661x over previous
import jax
import jax.numpy as jnp
from jax import lax
from jax.experimental import pallas as pl
from jax.experimental.pallas import tpu as pltpu
from jax.experimental.pallas import tpu_sc as plsc

_NUM_ATOMS = 320000
_FEA = 128
_NUM_SEG = 10000
_NC = 2
_NS = 16
_COLS = _FEA // _NC
_APT = _NUM_ATOMS // _NS
_CHUNK = 80
_NCHUNK = _APT // _CHUNK
_ROWS_PT = _NUM_SEG // _NS
_ZROWS = 125


def _body(feas, owner, out, rbuf, obuf, zbuf, acc):
    c = lax.axis_index("c")
    s = lax.axis_index("s")
    col0 = c * _COLS
    base = s * _APT

    def _zrow(i, carry):
        for j in range(_COLS // 16):
            zbuf[i, pl.ds(j * 16, 16)] = jnp.zeros((16,), jnp.float32)
        return carry

    lax.fori_loop(0, _ZROWS, _zrow, 0)
    r0 = s * _ROWS_PT
    for j in range(_ROWS_PT // _ZROWS):
        pltpu.sync_copy(zbuf, acc.at[pl.ds(r0 + j * _ZROWS, _ZROWS), :])
    plsc.subcore_barrier()

    def _chunk(k, carry):
        a0 = base + k * _CHUNK
        pltpu.sync_copy(feas.at[pl.ds(a0, _CHUNK), pl.ds(col0, _COLS)], rbuf)
        pltpu.sync_copy(owner.at[pl.ds(a0, _CHUNK)], obuf)
        pltpu.sync_copy(rbuf, acc.at[obuf], add=True)
        return carry

    lax.fori_loop(0, _NCHUNK, _chunk, 0)
    plsc.subcore_barrier()

    pltpu.sync_copy(
        acc.at[pl.ds(r0, _ROWS_PT), :],
        out.at[pl.ds(r0, _ROWS_PT), pl.ds(col0, _COLS)],
    )


_pool = pl.kernel(
    _body,
    out_type=jax.ShapeDtypeStruct((_NUM_SEG, _FEA), jnp.float32),
    mesh=plsc.VectorSubcoreMesh(
        core_axis_name="c", subcore_axis_name="s", num_cores=_NC,
        num_subcores=_NS,
    ),
    scratch_types=[
        pltpu.VMEM((_CHUNK, _COLS), jnp.float32),
        pltpu.VMEM((_CHUNK,), jnp.int32),
        pltpu.VMEM((_ZROWS, _COLS), jnp.float32),
        pltpu.VMEM_SHARED((_NUM_SEG, _COLS), jnp.float32),
    ],
    compiler_params=pltpu.CompilerParams(use_tc_tiling_on_sc=False),
)


@jax.jit
def kernel(atom_feas, atom_owner):
    return _pool(atom_feas, atom_owner.astype(jnp.int32))


# speedup vs baseline: 6.4761x; 2.4291x over previous
import jax
import jax.numpy as jnp
from jax import lax
from jax.experimental import pallas as pl
from jax.experimental.pallas import tpu as pltpu
from jax.experimental.pallas import tpu_sc as plsc

_NUM_ATOMS = 320000
_FEA = 128
_NUM_SEG = 10000
_NC = 2
_NS = 16
_COLS = _FEA // _NC
_APT = _NUM_ATOMS // _NS
_CHUNK = 80
_NCHUNK = _APT // _CHUNK
_ROWS_PT = _NUM_SEG // _NS
_ZROWS = 125


def _body(feas, owner, out, rb0, rb1, ob0, ob1, zbuf, acc,
          lsem0, lsem1, osem0, osem1):
    c = lax.axis_index("c")
    s = lax.axis_index("s")
    col0 = c * _COLS
    base = s * _APT
    rb = (rb0, rb1)
    ob = (ob0, ob1)
    lsem = (lsem0, lsem1)
    osem = (osem0, osem1)

    def _rows_at(a0):
        return feas.at[pl.ds(a0, _CHUNK), pl.ds(col0, _COLS)]

    def _own_at(a0):
        return owner.at[pl.ds(a0, _CHUNK)]

    for b in range(2):
        pltpu.async_copy(_rows_at(base + b * _CHUNK), rb[b], lsem[b])
        pltpu.async_copy(_own_at(base + b * _CHUNK), ob[b], osem[b])

    def _zrow(i, carry):
        for j in range(_COLS // 16):
            zbuf[i, pl.ds(j * 16, 16)] = jnp.zeros((16,), jnp.float32)
        return carry

    lax.fori_loop(0, _ZROWS, _zrow, 0)
    r0 = s * _ROWS_PT
    for j in range(_ROWS_PT // _ZROWS):
        pltpu.sync_copy(zbuf, acc.at[pl.ds(r0 + j * _ZROWS, _ZROWS), :])
    plsc.subcore_barrier()

    def _pair(i, carry):
        for b in range(2):
            k = 2 * i + b
            a0 = base + k * _CHUNK
            pltpu.make_async_copy(_rows_at(a0), rb[b], lsem[b]).wait()
            pltpu.make_async_copy(_own_at(a0), ob[b], osem[b]).wait()
            pltpu.sync_copy(rb[b], acc.at[ob[b]], add=True)
            an = base + jnp.minimum(k + 2, _NCHUNK - 1) * _CHUNK
            pltpu.async_copy(_rows_at(an), rb[b], lsem[b])
            pltpu.async_copy(_own_at(an), ob[b], osem[b])
        return carry

    lax.fori_loop(0, _NCHUNK // 2, _pair, 0)
    for b in range(2):
        pltpu.make_async_copy(_rows_at(base), rb[b], lsem[b]).wait()
        pltpu.make_async_copy(_own_at(base), ob[b], osem[b]).wait()
    plsc.subcore_barrier()

    pltpu.sync_copy(
        acc.at[pl.ds(r0, _ROWS_PT), :],
        out.at[pl.ds(r0, _ROWS_PT), pl.ds(col0, _COLS)],
    )


_pool = pl.kernel(
    _body,
    out_type=jax.ShapeDtypeStruct((_NUM_SEG, _FEA), jnp.float32),
    mesh=plsc.VectorSubcoreMesh(
        core_axis_name="c", subcore_axis_name="s", num_cores=_NC,
        num_subcores=_NS,
    ),
    scratch_types=[
        pltpu.VMEM((_CHUNK, _COLS), jnp.float32),
        pltpu.VMEM((_CHUNK, _COLS), jnp.float32),
        pltpu.VMEM((_CHUNK,), jnp.int32),
        pltpu.VMEM((_CHUNK,), jnp.int32),
        pltpu.VMEM((_ZROWS, _COLS), jnp.float32),
        pltpu.VMEM_SHARED((_NUM_SEG, _COLS), jnp.float32),
        pltpu.SemaphoreType.DMA,
        pltpu.SemaphoreType.DMA,
        pltpu.SemaphoreType.DMA,
        pltpu.SemaphoreType.DMA,
    ],
    compiler_params=pltpu.CompilerParams(use_tc_tiling_on_sc=False),
)


@jax.jit
def kernel(atom_feas, atom_owner):
    return _pool(atom_feas, atom_owner.astype(jnp.int32))


# speedup vs baseline: 7.3002x; 1.1273x over previous
import jax
import jax.numpy as jnp
from jax import lax
from jax.experimental import pallas as pl
from jax.experimental.pallas import tpu as pltpu
from jax.experimental.pallas import tpu_sc as plsc

_NUM_ATOMS = 320000
_FEA = 128
_NUM_SEG = 10000
_NC = 2
_NS = 16
_COLS = _FEA // _NC
_APT = _NUM_ATOMS // _NS
_CHUNK = 80
_NCHUNK = _APT // _CHUNK
_ROWS_PT = _NUM_SEG // _NS
_ZROWS = 125


def _body(feas, owner, out, rb0, rb1, rb2, rb3, ob0, ob1, ob2, ob3, zbuf,
          acc, lsem0, lsem1, lsem2, lsem3, osem0, osem1, osem2, osem3,
          ssem0, ssem1, ssem2, ssem3):
    c = lax.axis_index("c")
    s = lax.axis_index("s")
    col0 = c * _COLS
    base = s * _APT
    rb = (rb0, rb1, rb2, rb3)
    ob = (ob0, ob1, ob2, ob3)
    lsem = (lsem0, lsem1, lsem2, lsem3)
    osem = (osem0, osem1, osem2, osem3)
    ssem = (ssem0, ssem1, ssem2, ssem3)

    def _rows_at(a0):
        return feas.at[pl.ds(a0, _CHUNK), pl.ds(col0, _COLS)]

    def _own_at(a0):
        return owner.at[pl.ds(a0, _CHUNK)]

    def _issue_loads(k, b):
        a0 = base + k * _CHUNK
        pltpu.async_copy(_rows_at(a0), rb[b], lsem[b])
        pltpu.async_copy(_own_at(a0), ob[b], osem[b])

    def _wait_loads(b):
        pltpu.make_async_copy(_rows_at(base), rb[b], lsem[b]).wait()
        pltpu.make_async_copy(_own_at(base), ob[b], osem[b]).wait()

    def _start_scatter(b):
        pltpu.make_async_copy(rb[b], acc.at[ob[b]], ssem[b]).start(add=True)

    def _wait_scatter(b):
        pltpu.make_async_copy(rb[b], acc.at[ob[b]], ssem[b]).wait()

    for b in range(2):
        _issue_loads(b, b)

    def _zrow(i, carry):
        for j in range(_COLS // 16):
            zbuf[i, pl.ds(j * 16, 16)] = jnp.zeros((16,), jnp.float32)
        return carry

    lax.fori_loop(0, _ZROWS, _zrow, 0)
    r0 = s * _ROWS_PT
    for j in range(_ROWS_PT // _ZROWS):
        pltpu.sync_copy(zbuf, acc.at[pl.ds(r0 + j * _ZROWS, _ZROWS), :])
    plsc.subcore_barrier()

    for k in range(2):
        _wait_loads(k)
        _start_scatter(k)
        _issue_loads(k + 2, k + 2)

    def _quad(i, carry):
        for j in range(4):
            k = 2 + 4 * i + j
            b = (2 + j) % 4
            _wait_loads(b)
            _start_scatter(b)
            bn = j
            _wait_scatter(bn)
            kn = jnp.minimum(k + 2, _NCHUNK - 1)
            _issue_loads(kn, bn)
        return carry

    lax.fori_loop(0, (_NCHUNK - 2) // 4, _quad, 0)
    for b in range(2):
        _wait_loads(b + 2)
        _wait_scatter(b)
    plsc.subcore_barrier()

    pltpu.sync_copy(
        acc.at[pl.ds(r0, _ROWS_PT), :],
        out.at[pl.ds(r0, _ROWS_PT), pl.ds(col0, _COLS)],
    )


_pool = pl.kernel(
    _body,
    out_type=jax.ShapeDtypeStruct((_NUM_SEG, _FEA), jnp.float32),
    mesh=plsc.VectorSubcoreMesh(
        core_axis_name="c", subcore_axis_name="s", num_cores=_NC,
        num_subcores=_NS,
    ),
    scratch_types=(
        [pltpu.VMEM((_CHUNK, _COLS), jnp.float32)] * 4
        + [pltpu.VMEM((_CHUNK,), jnp.int32)] * 4
        + [
            pltpu.VMEM((_ZROWS, _COLS), jnp.float32),
            pltpu.VMEM_SHARED((_NUM_SEG, _COLS), jnp.float32),
        ]
        + [pltpu.SemaphoreType.DMA] * 12
    ),
    compiler_params=pltpu.CompilerParams(use_tc_tiling_on_sc=False),
)


@jax.jit
def kernel(atom_feas, atom_owner):
    return _pool(atom_feas, atom_owner.astype(jnp.int32))


# speedup vs baseline: 7.3548x; 1.0075x over previous
import jax
import jax.numpy as jnp
from jax import lax
from jax.experimental import pallas as pl
from jax.experimental.pallas import tpu as pltpu
from jax.experimental.pallas import tpu_sc as plsc

_NUM_ATOMS = 320000
_FEA = 128
_NUM_SEG = 10000
_NC = 2
_NS = 16
_COLS = _FEA // _NC
_APT = _NUM_ATOMS // _NS
_CHUNK = 80
_NCHUNK = _APT // _CHUNK
_ROWS_PT = _NUM_SEG // _NS
_ZROWS = 125


def _body(feas, owner2, out, rb0, rb1, rb2, rb3, obig, zbuf,
          acc, bsem, lsem0, lsem1, lsem2, lsem3,
          ssem0, ssem1, ssem2, ssem3):
    c = lax.axis_index("c")
    s = lax.axis_index("s")
    col0 = c * _COLS
    base = s * _APT
    rb = (rb0, rb1, rb2, rb3)
    lsem = (lsem0, lsem1, lsem2, lsem3)
    ssem = (ssem0, ssem1, ssem2, ssem3)

    def _rows_at(a0):
        return feas.at[pl.ds(a0, _CHUNK), pl.ds(col0, _COLS)]

    def _issue_load(k, b):
        pltpu.async_copy(_rows_at(base + k * _CHUNK), rb[b], lsem[b])

    def _wait_load(b):
        pltpu.make_async_copy(_rows_at(base), rb[b], lsem[b]).wait()

    def _start_scatter(k, b):
        pltpu.make_async_copy(
            rb[b], acc.at[obig.at[k]], ssem[b]).start(add=True)

    def _wait_scatter(b):
        pltpu.make_async_copy(rb[b], acc.at[obig.at[0]], ssem[b]).wait()

    pltpu.async_copy(owner2.at[pl.ds(s * _NCHUNK, _NCHUNK), :], obig, bsem)
    for b in range(2):
        _issue_load(b, b)

    def _zrow(i, carry):
        for j in range(_COLS // 16):
            zbuf[i, pl.ds(j * 16, 16)] = jnp.zeros((16,), jnp.float32)
        return carry

    lax.fori_loop(0, _ZROWS, _zrow, 0)
    r0 = s * _ROWS_PT
    for j in range(_ROWS_PT // _ZROWS):
        pltpu.sync_copy(zbuf, acc.at[pl.ds(r0 + j * _ZROWS, _ZROWS), :])
    pltpu.make_async_copy(
        owner2.at[pl.ds(s * _NCHUNK, _NCHUNK), :], obig, bsem).wait()
    plsc.subcore_barrier()

    for k in range(2):
        _wait_load(k)
        _start_scatter(k, k)
        _issue_load(k + 2, k + 2)

    def _quad(i, carry):
        for j in range(4):
            k = 2 + 4 * i + j
            b = (2 + j) % 4
            _wait_load(b)
            _start_scatter(k, b)
            bn = j
            _wait_scatter(bn)
            kn = jnp.minimum(k + 2, _NCHUNK - 1)
            _issue_load(kn, bn)
        return carry

    lax.fori_loop(0, (_NCHUNK - 2) // 4, _quad, 0)
    for b in range(2):
        _wait_load(b + 2)
        _wait_scatter(b)
    plsc.subcore_barrier()

    pltpu.sync_copy(
        acc.at[pl.ds(r0, _ROWS_PT), :],
        out.at[pl.ds(r0, _ROWS_PT), pl.ds(col0, _COLS)],
    )


_pool = pl.kernel(
    _body,
    out_type=jax.ShapeDtypeStruct((_NUM_SEG, _FEA), jnp.float32),
    mesh=plsc.VectorSubcoreMesh(
        core_axis_name="c", subcore_axis_name="s", num_cores=_NC,
        num_subcores=_NS,
    ),
    scratch_types=(
        [pltpu.VMEM((_CHUNK, _COLS), jnp.float32)] * 4
        + [
            pltpu.VMEM((_NCHUNK, _CHUNK), jnp.int32),
            pltpu.VMEM((_ZROWS, _COLS), jnp.float32),
            pltpu.VMEM_SHARED((_NUM_SEG, _COLS), jnp.float32),
        ]
        + [pltpu.SemaphoreType.DMA] * 9
    ),
    compiler_params=pltpu.CompilerParams(use_tc_tiling_on_sc=False),
)


@jax.jit
def kernel(atom_feas, atom_owner):
    owner2 = atom_owner.astype(jnp.int32).reshape(
        _NUM_ATOMS // _CHUNK, _CHUNK)
    return _pool(atom_feas, owner2)


# speedup vs baseline: 9.2148x; 1.2529x over previous
import jax
import jax.numpy as jnp
from jax import lax
from jax.experimental import pallas as pl
from jax.experimental.pallas import tpu as pltpu
from jax.experimental.pallas import tpu_sc as plsc

_NUM_ATOMS = 320000
_FEA = 128
_NUM_SEG = 10000
_NC = 2
_NS = 16
_COLS = _FEA // _NC
_APT = _NUM_ATOMS // _NS
_CHUNK = 80
_NCHUNK = _APT // _CHUNK
_ROWS_PT = _NUM_SEG // _NS
_ZROWS = 125


def _body(feas, owner2, out, rb0, rb1, rb2, rb3, rb4, rb5, rb6, rb7,
          obig, zbuf, acc, bsem,
          lsem0, lsem1, lsem2, lsem3, lsem4, lsem5, lsem6, lsem7,
          ssem0, ssem1, ssem2, ssem3, ssem4, ssem5, ssem6, ssem7):
    c = lax.axis_index("c")
    s = lax.axis_index("s")
    col0 = c * _COLS
    base = s * _APT
    rb = (rb0, rb1, rb2, rb3, rb4, rb5, rb6, rb7)
    lsem = (lsem0, lsem1, lsem2, lsem3, lsem4, lsem5, lsem6, lsem7)
    ssem = (ssem0, ssem1, ssem2, ssem3, ssem4, ssem5, ssem6, ssem7)

    def _rows_at(a0):
        return feas.at[pl.ds(a0, _CHUNK), pl.ds(col0, _COLS)]

    def _issue_load(k, b):
        pltpu.async_copy(_rows_at(base + k * _CHUNK), rb[b], lsem[b])

    def _wait_load(b):
        pltpu.make_async_copy(_rows_at(base), rb[b], lsem[b]).wait()

    def _start_scatter(k, b):
        pltpu.make_async_copy(
            rb[b], acc.at[obig.at[k]], ssem[b]).start(add=True)

    def _wait_scatter(b):
        pltpu.make_async_copy(rb[b], acc.at[obig.at[0]], ssem[b]).wait()

    pltpu.async_copy(owner2.at[pl.ds(s * _NCHUNK, _NCHUNK), :], obig, bsem)
    for b in range(6):
        _issue_load(b, b)

    def _zrow(i, carry):
        for j in range(_COLS // 16):
            zbuf[i, pl.ds(j * 16, 16)] = jnp.zeros((16,), jnp.float32)
        return carry

    lax.fori_loop(0, _ZROWS, _zrow, 0)
    r0 = s * _ROWS_PT
    for j in range(_ROWS_PT // _ZROWS):
        pltpu.sync_copy(zbuf, acc.at[pl.ds(r0 + j * _ZROWS, _ZROWS), :])
    pltpu.make_async_copy(
        owner2.at[pl.ds(s * _NCHUNK, _NCHUNK), :], obig, bsem).wait()
    plsc.subcore_barrier()

    for k in range(2):
        _wait_load(k)
        _start_scatter(k, k)
        _issue_load(k + 6, k + 6)

    def _oct(i, carry):
        for j in range(8):
            k = 2 + 8 * i + j
            b = (2 + j) % 8
            _wait_load(b)
            _start_scatter(k, b)
            bn = j
            _wait_scatter(bn)
            kn = jnp.minimum(k + 6, _NCHUNK - 1)
            _issue_load(kn, bn)
        return carry

    lax.fori_loop(0, (_NCHUNK - 2) // 8, _oct, 0)
    for b in range(6):
        _wait_load(b + 2)
    for b in range(2):
        _wait_scatter(b)
    plsc.subcore_barrier()

    pltpu.sync_copy(
        acc.at[pl.ds(r0, _ROWS_PT), :],
        out.at[pl.ds(r0, _ROWS_PT), pl.ds(col0, _COLS)],
    )


_pool = pl.kernel(
    _body,
    out_type=jax.ShapeDtypeStruct((_NUM_SEG, _FEA), jnp.float32),
    mesh=plsc.VectorSubcoreMesh(
        core_axis_name="c", subcore_axis_name="s", num_cores=_NC,
        num_subcores=_NS,
    ),
    scratch_types=(
        [pltpu.VMEM((_CHUNK, _COLS), jnp.float32)] * 8
        + [
            pltpu.VMEM((_NCHUNK, _CHUNK), jnp.int32),
            pltpu.VMEM((_ZROWS, _COLS), jnp.float32),
            pltpu.VMEM_SHARED((_NUM_SEG, _COLS), jnp.float32),
        ]
        + [pltpu.SemaphoreType.DMA] * 17
    ),
    compiler_params=pltpu.CompilerParams(use_tc_tiling_on_sc=False),
)


@jax.jit
def kernel(atom_feas, atom_owner):
    owner2 = atom_owner.astype(jnp.int32).reshape(
        _NUM_ATOMS // _CHUNK, _CHUNK)
    return _pool(atom_feas, owner2)
